# Initial kernel scaffold; baseline (speedup 1.0000x reference)
import jax
import jax.numpy as jnp
from jax.experimental import pallas as pl


def kernel(mix, falsetto, breathy, pharyngeal, vibrato, glissando, mix_w, falsetto_w, breathy_w, pharyngeal_w, vibrato_w, glissando_w):
    raise NotImplementedError("write your pallas kernel here")


# speedup vs baseline: 16.6900x; 16.6900x over previous
import functools

import jax
import jax.numpy as jnp
from jax import lax
from jax.experimental import pallas as pl
from jax.experimental.pallas import tpu as pltpu
from jax.experimental.pallas import tpu_sc as plsc

H = 256
NCOMB = 729
NCOMB_PAD = 736
SCALE = 16.0

NC = 2
NS = 16
NW = NC * NS
K = 128


def _prep_body(m_ref, f_ref, b_ref, p_ref, v_ref, g_ref,
               mw_ref, fw_ref, bw_ref, pw_ref, vw_ref, gw_ref,
               cidx_ref, tab_ref):
    cidx_ref[...] = (m_ref[...] + 3 * f_ref[...] + 9 * b_ref[...]
                     + 27 * p_ref[...] + 81 * v_ref[...] + 243 * g_ref[...])

    c = lax.broadcasted_iota(jnp.int32, (NCOMB_PAD, H), 0)

    def pick(w_ref, digit):
        w = w_ref[...]
        return jnp.where(digit == 0, w[0:1, :],
                         jnp.where(digit == 1, w[1:2, :], w[2:3, :]))

    acc = pick(mw_ref, c % 3) * SCALE
    acc = acc + pick(fw_ref, (c // 3) % 3) * SCALE
    acc = acc + pick(bw_ref, (c // 9) % 3) * SCALE
    acc = acc + pick(pw_ref, (c // 27) % 3) * SCALE
    acc = acc + pick(vw_ref, (c // 81) % 3) * SCALE
    acc = acc + pick(gw_ref, (c // 243) % 3) * SCALE
    tab_ref[...] = acc


def _make_gather_kernel(P):
    PPT = P // NW
    NCH = PPT // K
    mesh = plsc.VectorSubcoreMesh(core_axis_name="c", subcore_axis_name="s")

    @functools.partial(
        pl.kernel,
        mesh=mesh,
        out_type=jax.ShapeDtypeStruct((P, H), jnp.float32),
        scratch_types=[
            pltpu.VMEM((NCH, K), jnp.int32),
            pltpu.VMEM((2, K, H), jnp.float32),
            pltpu.SemaphoreType.DMA,
            pltpu.SemaphoreType.DMA,
            pltpu.SemaphoreType.DMA,
        ],
    )
    def gather_kernel(cidx_hbm, tab_hbm, out_hbm, cidx_v, rows_v,
                      gsem0, gsem1, wsem):
        wid = lax.axis_index("s") * NC + lax.axis_index("c")
        base = wid * PPT
        pltpu.sync_copy(cidx_hbm.at[pl.ds(wid * NCH, NCH)], cidx_v)
        gsems = (gsem0, gsem1)

        def gcopy(j):
            return pltpu.make_async_copy(
                tab_hbm.at[cidx_v.at[j]], rows_v.at[j % 2], gsems[j % 2])

        def wcopy(j):
            return pltpu.make_async_copy(
                rows_v.at[j % 2], out_hbm.at[pl.ds(base + j * K, K)], wsem)

        gcopy(0).start()
        for j in range(NCH):
            if j + 1 < NCH:
                if j >= 1:
                    wcopy(j - 1).wait()
                gcopy(j + 1).start()
            gcopy(j).wait()
            wcopy(j).start()
        wcopy(NCH - 2).wait()
        wcopy(NCH - 1).wait()

    return gather_kernel


def kernel(mix, falsetto, breathy, pharyngeal, vibrato, glissando,
           mix_w, falsetto_w, breathy_w, pharyngeal_w, vibrato_w, glissando_w):
    B, L = mix.shape
    P = B * L
    idx = [x.reshape(P // K, K).astype(jnp.int32)
           for x in (mix, falsetto, breathy, pharyngeal, vibrato, glissando)]

    cidx, tab = pl.pallas_call(
        _prep_body,
        out_shape=(jax.ShapeDtypeStruct((P // K, K), jnp.int32),
                   jax.ShapeDtypeStruct((NCOMB_PAD, H), jnp.float32)),
    )(*idx, mix_w, falsetto_w, breathy_w, pharyngeal_w, vibrato_w, glissando_w)

    out = _make_gather_kernel(P)(cidx, tab)
    return out.reshape(B, L, H)


# speedup vs baseline: 16.7363x; 1.0028x over previous
import functools

import jax
import jax.numpy as jnp
from jax import lax
from jax.experimental import pallas as pl
from jax.experimental.pallas import tpu as pltpu
from jax.experimental.pallas import tpu_sc as plsc

H = 256
NCOMB = 729
NCOMB_PAD = 736
SCALE = 16.0

NC = 2
NS = 16
NW = NC * NS
K = 128


def _prep_body(m_ref, f_ref, b_ref, p_ref, v_ref, g_ref,
               mw_ref, fw_ref, bw_ref, pw_ref, vw_ref, gw_ref,
               cidx_ref, tab_ref):
    cidx_ref[...] = (m_ref[...] + 3 * f_ref[...] + 9 * b_ref[...]
                     + 27 * p_ref[...] + 81 * v_ref[...] + 243 * g_ref[...])

    c = lax.broadcasted_iota(jnp.int32, (NCOMB_PAD, H), 0)

    def pick(w_ref, digit):
        w = w_ref[...]
        return jnp.where(digit == 0, w[0:1, :],
                         jnp.where(digit == 1, w[1:2, :], w[2:3, :]))

    acc = pick(mw_ref, c % 3) * SCALE
    acc = acc + pick(fw_ref, (c // 3) % 3) * SCALE
    acc = acc + pick(bw_ref, (c // 9) % 3) * SCALE
    acc = acc + pick(pw_ref, (c // 27) % 3) * SCALE
    acc = acc + pick(vw_ref, (c // 81) % 3) * SCALE
    acc = acc + pick(gw_ref, (c // 243) % 3) * SCALE
    tab_ref[...] = acc


def _make_gather_kernel(P):
    PPT = P // NW
    NCH = PPT // K
    NBUF = 3
    mesh = plsc.VectorSubcoreMesh(core_axis_name="c", subcore_axis_name="s")

    @functools.partial(
        pl.kernel,
        mesh=mesh,
        out_type=jax.ShapeDtypeStruct((P, H), jnp.float32),
        scratch_types=[
            pltpu.VMEM((NCH, K), jnp.int32),
            pltpu.VMEM((NBUF, K, H), jnp.float32),
            pltpu.SemaphoreType.DMA,
            pltpu.SemaphoreType.DMA,
            pltpu.SemaphoreType.DMA,
            pltpu.SemaphoreType.DMA,
        ],
    )
    def gather_kernel(cidx_hbm, tab_hbm, out_hbm, cidx_v, rows_v,
                      gsem0, gsem1, gsem2, wsem):
        wid = lax.axis_index("s") * NC + lax.axis_index("c")
        base = wid * PPT
        pltpu.sync_copy(cidx_hbm.at[pl.ds(wid * NCH, NCH)], cidx_v)
        gsems = (gsem0, gsem1, gsem2)

        def gcopy(j):
            return pltpu.make_async_copy(
                tab_hbm.at[cidx_v.at[j]], rows_v.at[j % NBUF],
                gsems[j % NBUF])

        def wcopy(j):
            return pltpu.make_async_copy(
                rows_v.at[j % NBUF], out_hbm.at[pl.ds(base + j * K, K)], wsem)

        for j in range(NBUF - 1):
            gcopy(j).start()
        for j in range(NCH):
            if j >= 1:
                wcopy(j - 1).wait()
            if j + NBUF - 1 < NCH:
                gcopy(j + NBUF - 1).start()
            gcopy(j).wait()
            wcopy(j).start()
        wcopy(NCH - 1).wait()

    return gather_kernel


def kernel(mix, falsetto, breathy, pharyngeal, vibrato, glissando,
           mix_w, falsetto_w, breathy_w, pharyngeal_w, vibrato_w, glissando_w):
    B, L = mix.shape
    P = B * L
    idx = [x.reshape(P // K, K).astype(jnp.int32)
           for x in (mix, falsetto, breathy, pharyngeal, vibrato, glissando)]

    cidx, tab = pl.pallas_call(
        _prep_body,
        out_shape=(jax.ShapeDtypeStruct((P // K, K), jnp.int32),
                   jax.ShapeDtypeStruct((NCOMB_PAD, H), jnp.float32)),
    )(*idx, mix_w, falsetto_w, breathy_w, pharyngeal_w, vibrato_w, glissando_w)

    out = _make_gather_kernel(P)(cidx, tab)
    return out.reshape(B, L, H)


# speedup vs baseline: 17.6305x; 1.0534x over previous
import functools

import jax
import jax.numpy as jnp
from jax import lax
from jax.experimental import pallas as pl
from jax.experimental.pallas import tpu as pltpu
from jax.experimental.pallas import tpu_sc as plsc

H = 256
NCOMB = 729
NCOMB_PAD = 768
SCALE = 16.0

NC = 2
NS = 16
NW = NC * NS
K = 128
RG = 8


def _prep_body(mw_ref, fw_ref, bw_ref, pw_ref, vw_ref, gw_ref, tab_ref):
    c = lax.broadcasted_iota(jnp.int32, (NCOMB_PAD, H), 0)

    def pick(w_ref, digit):
        w = w_ref[...]
        return jnp.where(digit == 0, w[0:1, :],
                         jnp.where(digit == 1, w[1:2, :], w[2:3, :]))

    acc = pick(mw_ref, c % 3) * SCALE
    acc = acc + pick(fw_ref, (c // 3) % 3) * SCALE
    acc = acc + pick(bw_ref, (c // 9) % 3) * SCALE
    acc = acc + pick(pw_ref, (c // 27) % 3) * SCALE
    acc = acc + pick(vw_ref, (c // 81) % 3) * SCALE
    acc = acc + pick(gw_ref, (c // 243) % 3) * SCALE
    tab_ref[...] = acc


def _make_gather_kernel(B, L):
    P = B * L
    CW = L // (NW // (B // RG))
    NCG = L // CW
    PPT = RG * CW
    NCH = PPT // K
    CPB = CW // K
    NBUF = 3
    mesh = plsc.VectorSubcoreMesh(core_axis_name="c", subcore_axis_name="s")

    @functools.partial(
        pl.kernel,
        mesh=mesh,
        out_type=jax.ShapeDtypeStruct((P, H), jnp.float32),
        scratch_types=[
            pltpu.VMEM((6, RG, CW), jnp.int32),
            pltpu.VMEM((NCH, K), jnp.int32),
            pltpu.VMEM((NBUF, K, H), jnp.float32),
            pltpu.SemaphoreType.DMA,
            pltpu.SemaphoreType.DMA,
            pltpu.SemaphoreType.DMA,
            pltpu.SemaphoreType.DMA,
        ],
    )
    def gather_kernel(m_h, f_h, b_h, p_h, v_h, g_h, tab_hbm, out_hbm,
                      idx_v, cidx_v, rows_v, gsem0, gsem1, gsem2, wsem):
        wid = lax.axis_index("s") * NC + lax.axis_index("c")
        rg = wid % (B // RG)
        cg = wid // (B // RG)
        rowbase = pl.multiple_of(rg * RG, 8)
        colbase = pl.multiple_of(cg * CW, 128)

        for i, src in enumerate((m_h, f_h, b_h, p_h, v_h, g_h)):
            pltpu.sync_copy(
                src.at[pl.ds(rowbase, RG), pl.ds(colbase, CW)], idx_v.at[i])

        for j in range(NCH):
            bi, hf = j // CPB, j % CPB
            for k in range(K // 16):
                sl = pl.ds(hf * K + 16 * k, 16)
                cidx_v[j, pl.ds(16 * k, 16)] = (
                    idx_v[0, bi, sl] + 3 * idx_v[1, bi, sl]
                    + 9 * idx_v[2, bi, sl] + 27 * idx_v[3, bi, sl]
                    + 81 * idx_v[4, bi, sl] + 243 * idx_v[5, bi, sl])

        gsems = (gsem0, gsem1, gsem2)

        def gcopy(j):
            return pltpu.make_async_copy(
                tab_hbm.at[cidx_v.at[j]], rows_v.at[j % NBUF],
                gsems[j % NBUF])

        def wcopy(j):
            bi, hf = j // CPB, j % CPB
            off = (rg * RG + bi) * L + cg * CW + hf * K
            return pltpu.make_async_copy(
                rows_v.at[j % NBUF],
                out_hbm.at[pl.ds(pl.multiple_of(off, 8), K)], wsem)

        for j in range(NBUF - 1):
            gcopy(j).start()
        for j in range(NCH):
            if j >= 1:
                wcopy(j - 1).wait()
            if j + NBUF - 1 < NCH:
                gcopy(j + NBUF - 1).start()
            gcopy(j).wait()
            wcopy(j).start()
        wcopy(NCH - 1).wait()

    return gather_kernel


def kernel(mix, falsetto, breathy, pharyngeal, vibrato, glissando,
           mix_w, falsetto_w, breathy_w, pharyngeal_w, vibrato_w, glissando_w):
    B, L = mix.shape
    idx = [x.astype(jnp.int32)
           for x in (mix, falsetto, breathy, pharyngeal, vibrato, glissando)]

    tab = pl.pallas_call(
        _prep_body,
        out_shape=jax.ShapeDtypeStruct((NCOMB_PAD, H), jnp.float32),
    )(mix_w, falsetto_w, breathy_w, pharyngeal_w, vibrato_w, glissando_w)

    out = _make_gather_kernel(B, L)(*idx, tab)
    return out.reshape(B, L, H)
